# baseline (speedup 1.0000x reference)
import functools

import jax
import jax.numpy as jnp
from jax import lax
from jax.experimental import pallas as pl
from jax.experimental.pallas import tpu as pltpu
from jax.experimental.pallas import tpu_sc as plsc

B = 16384
D = 64
L = 16
NC = 2
NS = 16
NW = NC * NS
BPW = B // NW
CHUNK = 128
NCHUNK = BPW // CHUNK
LAG = 8


def _mf_body(u_hbm, i_hbm, bu_hbm, bi_hbm, pt_hbm, qt_hbm, out_hbm,
             uidx_v, iidx_v, pv, qv, buv_v, biv_v, out_v, semp, semq, semb):
    wid = lax.axis_index("s") * NC + lax.axis_index("c")
    base = wid * BPW

    pltpu.sync_copy(u_hbm.at[wid], uidx_v)
    pltpu.sync_copy(i_hbm.at[wid], iidx_v)

    bias_copies = []
    for j in range(NCHUNK):
        sl = pl.ds(j * CHUNK, CHUNK)
        bias_copies.append(
            pltpu.async_copy(bu_hbm.at[uidx_v.at[j]], buv_v.at[sl], semb))
        bias_copies.append(
            pltpu.async_copy(bi_hbm.at[iidx_v.at[j]], biv_v.at[sl], semb))

    def drain(k):
        pltpu.make_async_copy(bu_hbm.at[pl.ds(0, BPW)], pv.at[k], semp).wait()
        pltpu.make_async_copy(bu_hbm.at[pl.ds(0, BPW)], qv.at[k], semq).wait()

    def fire(k, _):
        for j in range(NCHUNK):
            sl = pl.ds(j * CHUNK, CHUNK)
            pltpu.async_copy(pt_hbm.at[k].at[uidx_v.at[j]], pv.at[k, sl], semp)
            pltpu.async_copy(qt_hbm.at[k].at[iidx_v.at[j]], qv.at[k, sl], semq)
        @pl.when(k >= LAG)
        def _():
            drain(k - LAG)
        return _

    lax.fori_loop(0, D, fire, None)

    def tail(k, _):
        drain(k)
        return _

    lax.fori_loop(D - LAG, D, tail, None)
    for c in bias_copies:
        c.wait()

    def group(g, _):
        sl = pl.ds(g * L, L)
        acc = buv_v[sl] + biv_v[sl]
        for k in range(D):
            acc = acc + pv[k, sl] * qv[k, sl]
        out_v[sl] = acc
        return _

    lax.fori_loop(0, BPW // L, group, None)

    pltpu.sync_copy(out_v, out_hbm.at[pl.ds(base, BPW)])


_mf = functools.partial(
    pl.kernel,
    out_type=jax.ShapeDtypeStruct((B,), jnp.float32),
    mesh=plsc.VectorSubcoreMesh(core_axis_name="c", subcore_axis_name="s"),
    compiler_params=pltpu.CompilerParams(
        needs_layout_passes=False, use_tc_tiling_on_sc=False),
    scratch_types=[
        pltpu.VMEM((NCHUNK, CHUNK), jnp.int32),
        pltpu.VMEM((NCHUNK, CHUNK), jnp.int32),
        pltpu.VMEM((D, BPW), jnp.float32),
        pltpu.VMEM((D, BPW), jnp.float32),
        pltpu.VMEM((BPW,), jnp.float32),
        pltpu.VMEM((BPW,), jnp.float32),
        pltpu.VMEM((BPW,), jnp.float32),
        pltpu.SemaphoreType.DMA,
        pltpu.SemaphoreType.DMA,
        pltpu.SemaphoreType.DMA,
    ],
)(_mf_body)


@jax.jit
def kernel(u_idx, i_idx, mu, b_u, b_i, P, Q):
    u3 = u_idx.astype(jnp.int32).reshape(NW, NCHUNK, CHUNK)
    i3 = i_idx.astype(jnp.int32).reshape(NW, NCHUNK, CHUNK)
    out = _mf(u3, i3, b_u, b_i, P.T, Q.T)
    return out + mu
